# Initial kernel scaffold; baseline (speedup 1.0000x reference)
import jax
import jax.numpy as jnp
from jax.experimental import pallas as pl


def kernel(text_idxs, text_len, embedding_table):
    raise NotImplementedError("write your pallas kernel here")


# speedup vs baseline: 7.6897x; 7.6897x over previous
import functools

import jax
import jax.numpy as jnp
from jax import lax
from jax.experimental import pallas as pl
from jax.experimental.pallas import tpu as pltpu
from jax.experimental.pallas import tpu_sc as plsc

_LANES = 16


def _lane_sum_splat(v):
    lanes = lax.iota(jnp.int32, _LANES)
    dnums = lax.GatherDimensionNumbers(
        offset_dims=(), collapsed_slice_dims=(0,), start_index_map=(0,)
    )
    for sh in (1, 2, 4, 8):
        idx = (lanes ^ sh).reshape(_LANES, 1)
        shuf = lax.gather(
            v, idx, dnums, (1,), mode=lax.GatherScatterMode.PROMISE_IN_BOUNDS
        )
        v = v + shuf
    return v


@functools.cache
def _build(B, L, V, D):
    info = plsc.get_sparse_core_info()
    NC, NS = info.num_cores, info.num_subcores
    NW = NC * NS
    assert B % NW == 0
    BPW = B // NW
    C = 16
    assert BPW % C == 0
    NCH = BPW // C
    DV = D // _LANES

    mesh = plsc.VectorSubcoreMesh(core_axis_name="c", subcore_axis_name="s")

    @functools.partial(
        pl.kernel,
        mesh=mesh,
        out_type=jax.ShapeDtypeStruct((B, D), jnp.float32),
        scratch_types=[
            pltpu.VMEM((C * L,), jnp.int32),
            pltpu.VMEM((C * L, D), jnp.float32),
            pltpu.VMEM((C, D), jnp.float32),
            pltpu.SemaphoreType.DMA,
        ],
        compiler_params=pltpu.CompilerParams(use_tc_tiling_on_sc=False),
    )
    def bag_kernel(idx_hbm, table_hbm, out_hbm, idx_v, rows_v, out_v, sem):
        wid = lax.axis_index("s") * NC + lax.axis_index("c")
        base = wid * BPW

        def chunk(g, carry):
            r0 = base + g * C
            pltpu.sync_copy(idx_hbm.at[pl.ds(r0 * L, C * L)], idx_v)
            pltpu.async_copy(table_hbm.at[idx_v], rows_v, sem).wait()

            def bag(c, carry2):
                def row(l, acc):
                    r = c * L + l
                    return tuple(
                        acc[j] + rows_v[r, pl.ds(j * _LANES, _LANES)]
                        for j in range(DV)
                    )

                zero = jnp.zeros((_LANES,), jnp.float32)
                a = lax.fori_loop(0, L, row, (zero,) * DV)

                ssv = a[0] * a[0]
                for j in range(1, DV):
                    ssv = ssv + a[j] * a[j]
                x = _lane_sum_splat(ssv)
                xi = lax.bitcast_convert_type(x, jnp.int32)
                y = lax.bitcast_convert_type(
                    jnp.int32(0x5F3759DF) - (xi >> 1), jnp.float32
                )
                for _ in range(3):
                    y = y * (jnp.float32(1.5) - jnp.float32(0.5) * x * y * y)
                nrm = jnp.maximum(x * y, jnp.float32(1e-12))
                for j in range(DV):
                    out_v[c, pl.ds(j * _LANES, _LANES)] = a[j] / nrm
                return carry2

            lax.fori_loop(0, C, bag, 0)
            pltpu.sync_copy(out_v, out_hbm.at[pl.ds(r0, C)])
            return carry

        lax.fori_loop(0, NCH, chunk, 0)

    return bag_kernel


def kernel(text_idxs, text_len, embedding_table):
    B, L = text_idxs.shape
    V, D = embedding_table.shape
    k = _build(B, L, V, D)
    idx_flat = text_idxs.astype(jnp.int32).reshape(-1)
    return k(idx_flat, embedding_table)
